# baseline (speedup 1.0000x reference)
import functools
import jax
import jax.numpy as jnp
from jax import lax
from jax.experimental import pallas as pl
from jax.experimental.pallas import tpu as pltpu
from jax.experimental.pallas import tpu_sc as plsc

N_FEAT = 1000000
N_DIM = 32
BATCH = 16384

_info = plsc.get_sparse_core_info()
_NC = _info.num_cores
_NS = _info.num_subcores
_NW = _NC * _NS
_BPW = BATCH // _NW
_NQ = N_FEAT * N_DIM // 128
_NB = (N_FEAT + 127) // 128

_mesh = plsc.VectorSubcoreMesh(core_axis_name="c", subcore_axis_name="s")


@functools.partial(
    pl.kernel,
    mesh=_mesh,
    out_type=(
        jax.ShapeDtypeStruct((BATCH,), jnp.float32),
        jax.ShapeDtypeStruct((4, BATCH // 128, 8, 128), jnp.float32),
    ),
    scratch_types=[
        pltpu.VMEM((_BPW,), jnp.int32),
        pltpu.VMEM((_BPW,), jnp.int32),
        pltpu.VMEM((_BPW,), jnp.float32),
        pltpu.VMEM((_BPW, 128), jnp.float32),
        pltpu.VMEM((4, 4, 8, 128), jnp.float32),
        pltpu.SemaphoreType.DMA,
        pltpu.SemaphoreType.DMA,
    ],
    compiler_params=pltpu.CompilerParams(
        use_tc_tiling_on_sc=True, needs_layout_passes=False),
)
def _lookup(idx_hbm, vq_hbm, biasp_hbm, bias_out, out4,
            idx_v, blk_v, bias_v, fetched_v, colsT_v, sem_g, sem_o):
    wid = lax.axis_index("s") * _NC + lax.axis_index("c")
    base = wid * _BPW
    pltpu.sync_copy(idx_hbm.at[pl.ds(base, _BPW)], idx_v)
    lanes = lax.iota(jnp.int32, 16)
    nvec = _BPW // 16

    def bias_blk(jb):
        pos = jb * 16 + lanes
        i16 = plsc.load_gather(idx_v, [pos])
        plsc.store_scatter(blk_v, [pos], lax.shift_right_logical(i16, 7))

    pl.loop(0, nvec)(bias_blk)
    pltpu.async_copy(biasp_hbm.at[blk_v], fetched_v, sem_g).wait()

    def bias_ext(jb):
        pos = jb * 16 + lanes
        i16 = plsc.load_gather(idx_v, [pos])
        col = lax.bitwise_and(i16, 127)
        vals = plsc.load_gather(fetched_v, [pos, col])
        plsc.store_scatter(bias_v, [pos], vals)

    pl.loop(0, nvec)(bias_ext)

    def vect_blk(jb):
        pos = jb * 16 + lanes
        i16 = plsc.load_gather(idx_v, [pos])
        plsc.store_scatter(blk_v, [pos], lax.shift_right_logical(i16, 2))

    pl.loop(0, nvec)(vect_blk)
    pltpu.async_copy(vq_hbm.at[blk_v], fetched_v, sem_g).wait()

    zeros = jnp.zeros((16,), jnp.int32)
    for jb in range(nvec):
        pos = jb * 16 + lanes
        i16 = plsc.load_gather(idx_v, [pos])
        q32 = lax.bitwise_and(i16, 3) * 32
        cpos = (jb % 8) * 16 + lanes

        def dbody(d, _jb=jb, _q32=q32, _pos=pos, _cpos=cpos):
            vals = plsc.load_gather(fetched_v, [_pos, _q32 + d])
            plsc.store_scatter(
                colsT_v,
                [zeros + lax.div(d, 8), zeros + _jb // 8,
                 zeros + lax.rem(d, 8), _cpos],
                vals)

        pl.loop(0, N_DIM)(dbody)

    copies = []
    for g in range(4):
        for jj in range(4):
            copies.append(pltpu.async_copy(
                colsT_v.at[g, jj], out4.at[g, 4 * wid + jj], sem_o))
    for c in copies:
        c.wait()
    pltpu.sync_copy(bias_v, bias_out.at[pl.ds(base, _BPW)])


def kernel(index, vect, bias):
    idx = index.astype(jnp.int32)
    vq = vect.reshape(_NQ, 128)
    biasp = jnp.pad(bias[:, 0], (0, _NB * 128 - N_FEAT)).reshape(_NB, 128)
    bias_out, out4 = _lookup(idx, vq, biasp)
    return (bias_out.reshape(BATCH, 1),
            out4.transpose(1, 3, 0, 2).reshape(BATCH, N_DIM))
